# baseline (speedup 1.0000x reference)
import functools

import jax
import jax.numpy as jnp
from jax import lax
from jax.experimental import pallas as pl
from jax.experimental.pallas import tpu as pltpu
from jax.experimental.pallas import tpu_sc as plsc

N = 10000
E = 320000
D = 128
R = 3

CHUNK = 96
NW = 32
CH = 316
TOT_E = NW * CH * CHUNK
PAD_E = TOT_E - R * E
PADN = 10112
DUMMY = N
BM = 1000


def _mm_body(x_ref, w_ref, o_ref):
    o_ref[0] = jnp.dot(x_ref[...], w_ref[0], preferred_element_type=jnp.float32)


def _project(x, Ws):
    return pl.pallas_call(
        _mm_body,
        grid=(R, N // BM),
        in_specs=[
            pl.BlockSpec((BM, D), lambda r, i: (i, 0)),
            pl.BlockSpec((1, D, D), lambda r, i: (r, 0, 0)),
        ],
        out_specs=pl.BlockSpec((1, BM, D), lambda r, i: (r, i, 0)),
        out_shape=jax.ShapeDtypeStruct((R, N, D), jnp.float32),
    )(x, Ws)


def _comb_body(p_ref, o_ref):
    o_ref[...] = jnp.maximum(p_ref[0] + p_ref[1], 0.0)


def _combine(partials):
    return pl.pallas_call(
        _comb_body,
        grid=(N // BM,),
        in_specs=[pl.BlockSpec((2, BM, D), lambda i: (0, i, 0))],
        out_specs=pl.BlockSpec((BM, D), lambda i: (i, 0)),
        out_shape=jax.ShapeDtypeStruct((N, D), jnp.float32),
    )(partials)


_MESH = plsc.VectorSubcoreMesh(core_axis_name="c", subcore_axis_name="s")


@functools.partial(
    pl.kernel,
    out_type=jax.ShapeDtypeStruct((2, PADN, D), jnp.float32),
    mesh=_MESH,
    scratch_types=[
        pltpu.VMEM((CHUNK,), jnp.int32),
        pltpu.VMEM((CHUNK,), jnp.int32),
        pltpu.VMEM((CHUNK,), jnp.int32),
        pltpu.VMEM((CHUNK,), jnp.int32),
        pltpu.VMEM((CHUNK,), jnp.int32),
        pltpu.VMEM((CHUNK,), jnp.int32),
        pltpu.VMEM((CHUNK,), jnp.int32),
        pltpu.VMEM((CHUNK,), jnp.int32),
        pltpu.VMEM((CHUNK, D), jnp.float32),
        pltpu.VMEM((CHUNK, D), jnp.float32),
        pltpu.VMEM((CHUNK, D), jnp.float32),
        pltpu.VMEM((CHUNK, D), jnp.float32),
        pltpu.VMEM_SHARED((PADN, D), jnp.float32),
        pltpu.SemaphoreType.DMA,
        pltpu.SemaphoreType.DMA,
        pltpu.SemaphoreType.DMA,
        pltpu.SemaphoreType.DMA,
        pltpu.SemaphoreType.DMA,
        pltpu.SemaphoreType.DMA,
        pltpu.SemaphoreType.DMA,
        pltpu.SemaphoreType.DMA,
        pltpu.SemaphoreType.DMA,
        pltpu.SemaphoreType.DMA,
        pltpu.SemaphoreType.DMA,
        pltpu.SemaphoreType.DMA,
        pltpu.SemaphoreType.DMA,
        pltpu.SemaphoreType.DMA,
        pltpu.SemaphoreType.DMA,
        pltpu.SemaphoreType.DMA,
    ],
)
def _sc_edge(src_hbm, dst_hbm, h_hbm, z_hbm, out_hbm,
             sv0, sv1, sv2, sv3, dv0, dv1, dv2, dv3,
             rows0, rows1, rows2, rows3, acc,
             g0, g1, g2, g3, s0, s1, s2, s3,
             i0, i1, i2, i3, j0, j1, j2, j3):
    cid = lax.axis_index("c")
    sid = lax.axis_index("s")
    wid = sid * 2 + cid
    base_e = wid * (CH * CHUNK)

    svs = (sv0, sv1, sv2, sv3)
    dvs = (dv0, dv1, dv2, dv3)
    rows = (rows0, rows1, rows2, rows3)
    gsems = (g0, g1, g2, g3)
    ssems = (s0, s1, s2, s3)
    isems = (i0, i1, i2, i3)
    jsems = (j0, j1, j2, j3)

    rows_per_tile = PADN // 16
    pltpu.sync_copy(z_hbm, acc.at[pl.ds(sid * rows_per_tile, rows_per_tile)])
    plsc.subcore_barrier()

    def issue_src(b, j):
        pltpu.async_copy(src_hbm.at[pl.ds(base_e + j * CHUNK, CHUNK)],
                         svs[b], isems[b])

    def wait_src(b):
        pltpu.make_async_copy(src_hbm.at[pl.ds(0, CHUNK)], svs[b], isems[b]).wait()

    def issue_dst(b, j):
        pltpu.async_copy(dst_hbm.at[pl.ds(base_e + j * CHUNK, CHUNK)],
                         dvs[b], jsems[b])

    def wait_dst(b):
        pltpu.make_async_copy(dst_hbm.at[pl.ds(0, CHUNK)], dvs[b], jsems[b]).wait()

    def issue_gather(b):
        pltpu.async_copy(h_hbm.at[svs[b]], rows[b], gsems[b])

    def wait_gather(b):
        pltpu.make_async_copy(h_hbm.at[svs[b]], rows[b], gsems[b]).wait()

    def issue_scatter(b):
        pltpu.async_copy(rows[b], acc.at[dvs[b]], ssems[b], add=True)

    def wait_scatter(b):
        pltpu.make_async_copy(rows[b], acc.at[dvs[b]], ssems[b]).wait()

    for b in range(4):
        issue_src(b, b)
    issue_dst(0, 0)
    issue_dst(1, 1)
    wait_src(0)
    issue_gather(0)
    wait_src(1)
    issue_gather(1)

    def outer(g, _):
        for b in range(4):
            j = g * 4 + b
            b2 = (b + 2) % 4
            wait_gather(b)

            @pl.when(j >= 2)
            def _():
                wait_scatter(b2)

            wait_dst(b)
            issue_scatter(b)

            @pl.when(j + 2 < CH)
            def _():
                wait_src(b2)
                issue_gather(b2)
                issue_dst(b2, j + 2)

            @pl.when(j + 4 < CH)
            def _():
                issue_src(b, j + 4)

        return 0

    lax.fori_loop(0, CH // 4, outer, 0)
    wait_scatter((CH - 2) % 4)
    wait_scatter((CH - 1) % 4)

    plsc.subcore_barrier()
    out_rows = PADN // 16
    pltpu.sync_copy(
        acc.at[pl.ds(sid * out_rows, out_rows)],
        out_hbm.at[cid, pl.ds(sid * out_rows, out_rows), :],
    )


def kernel(x, edge_index_rel0, edge_index_rel1, edge_index_rel2,
           W_rel0, W_rel1, W_rel2):
    Ws = jnp.stack([W_rel0, W_rel1, W_rel2])
    h = _project(x, Ws).reshape(R * N, D)

    pad_src = jnp.zeros((PAD_E,), jnp.int32)
    pad_dst = jnp.full((PAD_E,), DUMMY, jnp.int32)
    src = jnp.concatenate(
        [edge_index_rel0[0], edge_index_rel1[0] + N, edge_index_rel2[0] + 2 * N,
         pad_src])
    dst = jnp.concatenate(
        [edge_index_rel0[1], edge_index_rel1[1], edge_index_rel2[1], pad_dst])

    zrows = jnp.zeros((PADN // 16, D), jnp.float32)
    partials = _sc_edge(src, dst, h, zrows)
    return _combine(partials)


# speedup vs baseline: 1.0346x; 1.0346x over previous
import functools

import jax
import jax.numpy as jnp
from jax import lax
from jax.experimental import pallas as pl
from jax.experimental.pallas import tpu as pltpu
from jax.experimental.pallas import tpu_sc as plsc

N = 10000
E = 320000
D = 128
R = 3

CHUNK = 128
NW = 32
CH = 237
TOT_E = NW * CH * CHUNK
PAD_E = TOT_E - R * E
PADN = 10112
DUMMY = N
BM = 1000


def _mm_body(x_ref, w_ref, o_ref):
    o_ref[0] = jnp.dot(x_ref[...], w_ref[0], preferred_element_type=jnp.float32)


def _project(x, Ws):
    return pl.pallas_call(
        _mm_body,
        grid=(R, N // BM),
        in_specs=[
            pl.BlockSpec((BM, D), lambda r, i: (i, 0)),
            pl.BlockSpec((1, D, D), lambda r, i: (r, 0, 0)),
        ],
        out_specs=pl.BlockSpec((1, BM, D), lambda r, i: (r, i, 0)),
        out_shape=jax.ShapeDtypeStruct((R, N, D), jnp.float32),
    )(x, Ws)


def _comb_body(p_ref, o_ref):
    o_ref[...] = jnp.maximum(p_ref[0] + p_ref[1], 0.0)


def _combine(partials):
    return pl.pallas_call(
        _comb_body,
        grid=(N // BM,),
        in_specs=[pl.BlockSpec((2, BM, D), lambda i: (0, i, 0))],
        out_specs=pl.BlockSpec((BM, D), lambda i: (i, 0)),
        out_shape=jax.ShapeDtypeStruct((N, D), jnp.float32),
    )(partials)


_MESH = plsc.VectorSubcoreMesh(core_axis_name="c", subcore_axis_name="s")


@functools.partial(
    pl.kernel,
    out_type=jax.ShapeDtypeStruct((2, PADN, D), jnp.float32),
    mesh=_MESH,
    scratch_types=[
        pltpu.VMEM((CHUNK,), jnp.int32),
        pltpu.VMEM((CHUNK,), jnp.int32),
        pltpu.VMEM((CHUNK,), jnp.int32),
        pltpu.VMEM((CHUNK,), jnp.int32),
        pltpu.VMEM((CHUNK,), jnp.int32),
        pltpu.VMEM((CHUNK,), jnp.int32),
        pltpu.VMEM((CHUNK, D), jnp.float32),
        pltpu.VMEM((CHUNK, D), jnp.float32),
        pltpu.VMEM((CHUNK, D), jnp.float32),
        pltpu.VMEM_SHARED((PADN, D), jnp.float32),
        pltpu.SemaphoreType.DMA,
        pltpu.SemaphoreType.DMA,
        pltpu.SemaphoreType.DMA,
        pltpu.SemaphoreType.DMA,
        pltpu.SemaphoreType.DMA,
        pltpu.SemaphoreType.DMA,
        pltpu.SemaphoreType.DMA,
        pltpu.SemaphoreType.DMA,
        pltpu.SemaphoreType.DMA,
        pltpu.SemaphoreType.DMA,
        pltpu.SemaphoreType.DMA,
        pltpu.SemaphoreType.DMA,
    ],
)
def _sc_edge(src_hbm, dst_hbm, h_hbm, z_hbm, out_hbm,
             sv0, sv1, sv2, dv0, dv1, dv2,
             rows0, rows1, rows2, acc,
             g0, g1, g2, s0, s1, s2,
             i0, i1, i2, j0, j1, j2):
    cid = lax.axis_index("c")
    sid = lax.axis_index("s")
    wid = sid * 2 + cid
    base_e = wid * (CH * CHUNK)

    svs = (sv0, sv1, sv2)
    dvs = (dv0, dv1, dv2)
    rows = (rows0, rows1, rows2)
    gsems = (g0, g1, g2)
    ssems = (s0, s1, s2)
    isems = (i0, i1, i2)
    jsems = (j0, j1, j2)

    rows_per_tile = PADN // 16
    pltpu.sync_copy(z_hbm, acc.at[pl.ds(sid * rows_per_tile, rows_per_tile)])
    plsc.subcore_barrier()

    def issue_src(b, j):
        pltpu.async_copy(src_hbm.at[pl.ds(base_e + j * CHUNK, CHUNK)],
                         svs[b], isems[b])

    def wait_src(b):
        pltpu.make_async_copy(src_hbm.at[pl.ds(0, CHUNK)], svs[b], isems[b]).wait()

    def issue_dst(b, j):
        pltpu.async_copy(dst_hbm.at[pl.ds(base_e + j * CHUNK, CHUNK)],
                         dvs[b], jsems[b])

    def wait_dst(b):
        pltpu.make_async_copy(dst_hbm.at[pl.ds(0, CHUNK)], dvs[b], jsems[b]).wait()

    def issue_gather(b):
        pltpu.async_copy(h_hbm.at[svs[b]], rows[b], gsems[b])

    def wait_gather(b):
        pltpu.make_async_copy(h_hbm.at[svs[b]], rows[b], gsems[b]).wait()

    def issue_scatter(b):
        pltpu.async_copy(rows[b], acc.at[dvs[b]], ssems[b], add=True)

    def wait_scatter(b):
        pltpu.make_async_copy(rows[b], acc.at[dvs[b]], ssems[b]).wait()

    for b in range(3):
        issue_src(b, b)
    issue_dst(0, 0)
    issue_dst(1, 1)
    wait_src(0)
    issue_gather(0)
    wait_src(1)
    issue_gather(1)

    def outer(g, _):
        for b in range(3):
            j = g * 3 + b
            bp = (b + 2) % 3
            wait_gather(b)
            wait_dst(b)
            issue_scatter(b)

            @pl.when(j >= 1)
            def _():
                wait_scatter(bp)

            @pl.when(j + 2 < CH)
            def _():
                wait_src(bp)
                issue_gather(bp)
                issue_dst(bp, j + 2)

            @pl.when(j + 3 < CH)
            def _():
                issue_src(b, j + 3)

        return 0

    lax.fori_loop(0, CH // 3, outer, 0)
    wait_scatter((CH - 1) % 3)

    plsc.subcore_barrier()
    out_rows = PADN // 16
    pltpu.sync_copy(
        acc.at[pl.ds(sid * out_rows, out_rows)],
        out_hbm.at[cid, pl.ds(sid * out_rows, out_rows), :],
    )


def kernel(x, edge_index_rel0, edge_index_rel1, edge_index_rel2,
           W_rel0, W_rel1, W_rel2):
    Ws = jnp.stack([W_rel0, W_rel1, W_rel2])
    h = _project(x, Ws).reshape(R * N, D)

    pad_src = jnp.zeros((PAD_E,), jnp.int32)
    pad_dst = jnp.full((PAD_E,), DUMMY, jnp.int32)
    src = jnp.concatenate(
        [edge_index_rel0[0], edge_index_rel1[0] + N, edge_index_rel2[0] + 2 * N,
         pad_src])
    dst = jnp.concatenate(
        [edge_index_rel0[1], edge_index_rel1[1], edge_index_rel2[1], pad_dst])

    zrows = jnp.zeros((PADN // 16, D), jnp.float32)
    partials = _sc_edge(src, dst, h, zrows)
    return _combine(partials)


# speedup vs baseline: 1.9963x; 1.9295x over previous
import functools

import jax
import jax.numpy as jnp
from jax import lax
from jax.experimental import pallas as pl
from jax.experimental.pallas import tpu as pltpu
from jax.experimental.pallas import tpu_sc as plsc

N = 10000
E = 320000
D = 128
R = 3

CHUNK = 128
QUADS = E // (4 * CHUNK)
C0Q = 412
PADN = 10112
BM = 1000


def _mm_body(x_ref, w_ref, o_ref):
    o_ref[...] = jnp.dot(x_ref[...], w_ref[...], preferred_element_type=jnp.float32)


def _project(x, W):
    return pl.pallas_call(
        _mm_body,
        grid=(N // BM,),
        in_specs=[
            pl.BlockSpec((BM, D), lambda i: (i, 0)),
            pl.BlockSpec((D, D), lambda i: (0, 0)),
        ],
        out_specs=pl.BlockSpec((BM, D), lambda i: (i, 0)),
        out_shape=jax.ShapeDtypeStruct((N, D), jnp.float32),
    )(x, W)


def _comb_body(p_ref, o_ref):
    o_ref[...] = jnp.maximum(p_ref[0] + p_ref[1], 0.0)


def _combine(partials):
    return pl.pallas_call(
        _comb_body,
        grid=(N // BM,),
        in_specs=[pl.BlockSpec((2, BM, D), lambda i: (0, i, 0))],
        out_specs=pl.BlockSpec((BM, D), lambda i: (i, 0)),
        out_shape=jax.ShapeDtypeStruct((N, D), jnp.float32),
    )(partials)


_MESH = plsc.VectorSubcoreMesh(core_axis_name="c", subcore_axis_name="s")


@functools.partial(
    pl.kernel,
    out_type=jax.ShapeDtypeStruct((2, PADN, D), jnp.float32),
    mesh=_MESH,
    scratch_types=[
        pltpu.VMEM((CHUNK,), jnp.int32),
        pltpu.VMEM((CHUNK,), jnp.int32),
        pltpu.VMEM((CHUNK,), jnp.int32),
        pltpu.VMEM((CHUNK,), jnp.int32),
        pltpu.VMEM((CHUNK,), jnp.int32),
        pltpu.VMEM((CHUNK,), jnp.int32),
        pltpu.VMEM((CHUNK,), jnp.int32),
        pltpu.VMEM((CHUNK,), jnp.int32),
        pltpu.VMEM((CHUNK, D), jnp.float32),
        pltpu.VMEM((CHUNK, D), jnp.float32),
        pltpu.VMEM_SHARED((PADN, D), jnp.float32),
        pltpu.SemaphoreType.DMA,
        pltpu.SemaphoreType.DMA,
        pltpu.SemaphoreType.DMA,
        pltpu.SemaphoreType.DMA,
        pltpu.SemaphoreType.DMA,
        pltpu.SemaphoreType.DMA,
    ],
)
def _sc_edge(e0_hbm, e1_hbm, e2_hbm, h0_hbm, h1_hbm, h2_hbm, z_hbm, out_hbm,
             sv0, sv1, sv2, sv3, dv0, dv1, dv2, dv3,
             rows0, rows1, acc,
             g0, g1, i0, i1, i2, i3):
    cid = lax.axis_index("c")
    sid = lax.axis_index("s")

    svs = (sv0, sv1, sv2, sv3)
    dvs = (dv0, dv1, dv2, dv3)
    rows = (rows0, rows1)
    gsems = (g0, g1)
    isems = (i0, i1, i2, i3)

    coreq = jnp.where(cid == 0, C0Q, QUADS - C0Q)
    corebase = jnp.where(cid == 0, 0, C0Q)
    q = coreq // 16
    rmd = coreq % 16
    myq = q + (sid < rmd).astype(jnp.int32)
    mystart = corebase + sid * q + jnp.minimum(sid, rmd)
    base_e = mystart * (4 * CHUNK)
    nch = myq * 4

    rows_per_tile = PADN // 16
    pltpu.sync_copy(z_hbm, acc.at[pl.ds(sid * rows_per_tile, rows_per_tile)])
    plsc.subcore_barrier()

    def run_rel(e_hbm, h_hbm):
        def issue_idx(b, j):
            off = base_e + j * CHUNK
            pltpu.async_copy(e_hbm.at[pl.ds(off, CHUNK)], svs[b], isems[b])
            pltpu.async_copy(e_hbm.at[pl.ds(E + off, CHUNK)], dvs[b], isems[b])

        def wait_idx(b):
            pltpu.make_async_copy(e_hbm.at[pl.ds(0, CHUNK)], svs[b], isems[b]).wait()
            pltpu.make_async_copy(e_hbm.at[pl.ds(0, CHUNK)], dvs[b], isems[b]).wait()

        def issue_gather(rb, b):
            pltpu.async_copy(h_hbm.at[svs[b]], rows[rb], gsems[rb])

        def wait_gather(rb, b):
            pltpu.make_async_copy(h_hbm.at[svs[b]], rows[rb], gsems[rb]).wait()

        for b in range(4):
            issue_idx(b, b)
        wait_idx(0)
        issue_gather(0, 0)
        wait_idx(1)
        issue_gather(1, 1)

        def outer(g, _):
            for b4 in range(4):
                j = g * 4 + b4
                rb = b4 % 2
                b2 = (b4 + 2) % 4
                wait_gather(rb, b4)
                pltpu.sync_copy(rows[rb], acc.at[dvs[b4]], add=True)

                @pl.when(j + 2 < nch)
                def _():
                    wait_idx(b2)
                    issue_gather(rb, b2)

                @pl.when(j + 4 < nch)
                def _():
                    issue_idx(b4, j + 4)

            return 0

        lax.fori_loop(0, myq, outer, 0)

    run_rel(e0_hbm, h0_hbm)
    run_rel(e1_hbm, h1_hbm)
    run_rel(e2_hbm, h2_hbm)

    plsc.subcore_barrier()
    pltpu.sync_copy(
        acc.at[pl.ds(sid * rows_per_tile, rows_per_tile)],
        out_hbm.at[cid, pl.ds(sid * rows_per_tile, rows_per_tile), :],
    )


def kernel(x, edge_index_rel0, edge_index_rel1, edge_index_rel2,
           W_rel0, W_rel1, W_rel2):
    h0 = _project(x, W_rel0)
    h1 = _project(x, W_rel1)
    h2 = _project(x, W_rel2)
    zrows = jnp.zeros((PADN // 16, D), jnp.float32)
    partials = _sc_edge(
        edge_index_rel0.reshape(-1), edge_index_rel1.reshape(-1),
        edge_index_rel2.reshape(-1), h0, h1, h2, zrows)
    return _combine(partials)
